# baseline (speedup 1.0000x reference)
import functools

import jax
import jax.numpy as jnp
import numpy as np
from jax import lax
from jax.experimental import pallas as pl
from jax.experimental.pallas import tpu as pltpu
from jax.experimental.pallas import tpu_sc as plsc

_B, _L, _D = 1024, 200, 32
_SAMPLE_NUM = 10
_LM1 = _L - 1
_NNEG = _LM1 * _SAMPLE_NUM
_POS_OFF = _NNEG + 2
_NIDX = _POS_OFF + _LM1 + 1


def _sample_idx_table(L, sample_num, seed=0):
    rng = np.random.RandomState(seed)
    all_idx = [
        rng.choice([j for j in range(L) if j != idx_], size=sample_num, replace=False)
        for idx_ in range(L)
    ]
    return np.stack(all_idx[1:], axis=0).astype(np.int32)


_NC = 2
_NS = 16
_NW = _NC * _NS
_BPW = _B // _NW


_mesh = plsc.VectorSubcoreMesh(core_axis_name="c", subcore_axis_name="s")


@functools.partial(
    pl.kernel,
    mesh=_mesh,
    out_type=(
        jax.ShapeDtypeStruct((_B, _LM1, _D), jnp.float32),
        jax.ShapeDtypeStruct((_B, _NNEG, _D), jnp.float32),
    ),
    scratch_types=[
        pltpu.VMEM((_NIDX,), jnp.int32),
        pltpu.VMEM((_NIDX, _D), jnp.float32),
        pltpu.SemaphoreType.DMA,
    ],
    compiler_params=pltpu.CompilerParams(use_tc_tiling_on_sc=False),
)
def _sc_sample(in_hbm, idx_hbm, pos_hbm, neg_hbm, idx_v, buf_v, sem):
    wid = lax.axis_index("s") * _NC + lax.axis_index("c")
    base = wid * _BPW

    def body(i, carry):
        b = base + i
        pltpu.sync_copy(idx_hbm.at[b], idx_v)
        pltpu.async_copy(in_hbm.at[idx_v], buf_v, sem).wait()
        pltpu.sync_copy(buf_v.at[pl.ds(0, _NNEG)], neg_hbm.at[b])
        pltpu.sync_copy(buf_v.at[pl.ds(_POS_OFF, _LM1)], pos_hbm.at[b])
        return carry

    lax.fori_loop(0, _BPW, body, 0)


def kernel(inputs):
    b, l, d = inputs.shape
    table = _sample_idx_table(l, _SAMPLE_NUM)
    per_batch = np.zeros((_NIDX,), np.int32)
    per_batch[:_NNEG] = table.reshape(-1)
    per_batch[_POS_OFF : _POS_OFF + _LM1] = np.arange(1, l, dtype=np.int32)
    idx_all = jnp.asarray(per_batch)[None, :] + (
        jnp.arange(b, dtype=jnp.int32) * l
    )[:, None]
    in_flat = inputs.reshape(b * l, d)
    pos, neg = _sc_sample(in_flat, idx_all)
    return pos, neg.reshape(b, _LM1, _SAMPLE_NUM, d)


# speedup vs baseline: 11.3693x; 11.3693x over previous
import functools

import jax
import jax.numpy as jnp
import numpy as np
from jax import lax
from jax.experimental import pallas as pl
from jax.experimental.pallas import tpu as pltpu
from jax.experimental.pallas import tpu_sc as plsc

_B, _L, _D = 1024, 200, 32
_SAMPLE_NUM = 10
_LM1 = _L - 1
_NNEG = _LM1 * _SAMPLE_NUM


def _sample_idx_table(L, sample_num, seed=0):
    rng = np.random.RandomState(seed)
    all_idx = [
        rng.choice([j for j in range(L) if j != idx_], size=sample_num, replace=False)
        for idx_ in range(L)
    ]
    return np.stack(all_idx[1:], axis=0).astype(np.int32)


def _routing_tables():
    flat = _sample_idx_table(_L, _SAMPLE_NUM).reshape(-1)
    counts = np.bincount(flat, minlength=_L).astype(np.int32)
    assert int(counts.max()) <= 31
    dtbl = np.zeros((_L, 32), np.int32)
    dtbl[:, 0] = counts
    fill = np.ones((_L,), np.int32)
    for k, j in enumerate(flat):
        dtbl[j, fill[j]] = k
        fill[j] += 1
    return dtbl


_DTBL = _routing_tables()

_NC = 2
_NS = 16
_NW = _NC * _NS

_mesh = plsc.VectorSubcoreMesh(core_axis_name="c", subcore_axis_name="s")


@functools.partial(
    pl.kernel,
    mesh=_mesh,
    out_type=(
        jax.ShapeDtypeStruct((_LM1, _D, _B), jnp.float32),
        jax.ShapeDtypeStruct((_NNEG, _D, _B), jnp.float32),
    ),
    scratch_types=[
        pltpu.VMEM((_L, 32), jnp.int32),
        pltpu.VMEM((_D, _B), jnp.float32),
    ],
    compiler_params=pltpu.CompilerParams(needs_layout_passes=False),
)
def _sc_route(x_hbm, dtbl_hbm, pos_hbm, neg_hbm, dtbl_v, slab_v):
    w = lax.axis_index("s") * _NC + lax.axis_index("c")
    pltpu.sync_copy(dtbl_hbm, dtbl_v)
    lo = (w * _L) // _NW
    hi = ((w + 1) * _L) // _NW
    lanes = lax.broadcasted_iota(jnp.int32, (16,), 0)

    def slab_body(j, carry):
        pltpu.sync_copy(x_hbm.at[j], slab_v)
        rv1 = dtbl_v[j, pl.ds(0, 16)]
        rv2 = dtbl_v[j, pl.ds(16, 16)]
        cnt = jnp.sum(jnp.where(lanes == 0, rv1, 0))

        @pl.when(j >= 1)
        def _():
            pltpu.sync_copy(slab_v, pos_hbm.at[j - 1])

        def dest_body(c, carry2):
            cc = c + 1
            rv = jnp.where(cc < 16, rv1, rv2)
            lane = jnp.where(cc < 16, cc, cc - 16)
            d = jnp.sum(jnp.where(lanes == lane, rv, 0))
            pltpu.sync_copy(slab_v, neg_hbm.at[d])
            return carry2

        lax.fori_loop(0, cnt, dest_body, 0)
        return carry

    lax.fori_loop(lo, hi, slab_body, 0)


def kernel(inputs):
    b, l, d = inputs.shape
    x_t = jnp.transpose(inputs, (1, 2, 0))
    pos_t, neg_t = _sc_route(x_t, jnp.asarray(_DTBL))
    pos = jnp.transpose(pos_t, (2, 0, 1))
    neg = jnp.transpose(
        neg_t.reshape(_LM1, _SAMPLE_NUM, d, b), (3, 0, 1, 2)
    )
    return pos, neg


# speedup vs baseline: 13.5518x; 1.1920x over previous
import functools

import jax
import jax.numpy as jnp
import numpy as np
from jax import lax
from jax.experimental import pallas as pl
from jax.experimental.pallas import tpu as pltpu
from jax.experimental.pallas import tpu_sc as plsc

_B, _L, _D = 1024, 200, 32
_SAMPLE_NUM = 10
_LM1 = _L - 1
_NNEG = _LM1 * _SAMPLE_NUM

_NC = 2
_NS = 16
_NW = _NC * _NS


def _sample_idx_table(L, sample_num, seed=0):
    rng = np.random.RandomState(seed)
    all_idx = [
        rng.choice([j for j in range(L) if j != idx_], size=sample_num, replace=False)
        for idx_ in range(L)
    ]
    return np.stack(all_idx[1:], axis=0).astype(np.int32)


def _routing_tables():
    flat = _sample_idx_table(_L, _SAMPLE_NUM).reshape(-1)
    counts = np.bincount(flat, minlength=_L).astype(np.int32)
    assert int(counts.max()) <= 31
    dtbl = np.zeros((_L, 32), np.int32)
    dtbl[:, 0] = counts
    fill = np.ones((_L,), np.int32)
    for k, j in enumerate(flat):
        dtbl[j, fill[j]] = k
        fill[j] += 1

    weight = counts + (np.arange(_L) >= 1)
    order = np.argsort(-weight, kind="stable")
    loads = np.zeros(_NW, np.int64)
    assign = [[] for _ in range(_NW)]
    for j in order:
        w = int(np.argmin(loads))
        loads[w] += int(weight[j])
        assign[w].append(int(j))
    max_slabs = max(len(a) for a in assign)
    assert max_slabs <= 15
    wtbl = np.zeros((_NW, 16), np.int32)
    for w, a in enumerate(assign):
        wtbl[w, 0] = len(a)
        wtbl[w, 1 : 1 + len(a)] = a
    return dtbl, wtbl


_DTBL, _WTBL = _routing_tables()

_mesh = plsc.VectorSubcoreMesh(core_axis_name="c", subcore_axis_name="s")


@functools.partial(
    pl.kernel,
    mesh=_mesh,
    out_type=(
        jax.ShapeDtypeStruct((_LM1, _D, _B), jnp.float32),
        jax.ShapeDtypeStruct((_NNEG, _D, _B), jnp.float32),
    ),
    scratch_types=[
        pltpu.VMEM((_L, 32), jnp.int32),
        pltpu.VMEM((_NW, 16), jnp.int32),
        pltpu.VMEM((2, _D, _B), jnp.float32),
        pltpu.SemaphoreType.DMA,
        pltpu.SemaphoreType.DMA,
        pltpu.SemaphoreType.DMA,
        pltpu.SemaphoreType.DMA,
    ],
    compiler_params=pltpu.CompilerParams(needs_layout_passes=False),
)
def _sc_route(
    x_hbm, dtbl_hbm, wtbl_hbm, pos_hbm, neg_hbm,
    dtbl_v, wtbl_v, slab_v, lsem0, lsem1, wsem0, wsem1,
):
    w = lax.axis_index("s") * _NC + lax.axis_index("c")
    pltpu.sync_copy(dtbl_hbm, dtbl_v)
    pltpu.sync_copy(wtbl_hbm, wtbl_v)
    lanes = lax.broadcasted_iota(jnp.int32, (16,), 0)
    wv = wtbl_v[w, pl.ds(0, 16)]

    def lane_of(vec, lane):
        return jnp.sum(jnp.where(lanes == lane, vec, 0))

    nslab = lane_of(wv, 0)
    j0 = lane_of(wv, 1)
    pltpu.async_copy(x_hbm.at[j0], slab_v.at[0], lsem0)

    def drain(sem, n):
        def one(_, c):
            pltpu.make_async_copy(slab_v.at[0], neg_hbm.at[0], sem).wait()
            return c

        lax.fori_loop(0, n, one, 0)

    def slab_body(si, prev_writes):
        p = si % 2
        j = lane_of(wv, si + 1)
        rv1 = dtbl_v[j, pl.ds(0, 16)]
        rv2 = dtbl_v[j, pl.ds(16, 16)]
        cnt = lane_of(rv1, 0)

        @pl.when(p == 0)
        def _():
            pltpu.make_async_copy(x_hbm.at[j], slab_v.at[0], lsem0).wait()

        @pl.when(p == 1)
        def _():
            pltpu.make_async_copy(x_hbm.at[j], slab_v.at[1], lsem1).wait()

        def dest_body(c, carry2):
            cc = c + 1
            rv = jnp.where(cc < 16, rv1, rv2)
            lane = jnp.where(cc < 16, cc, cc - 16)
            d = jnp.sum(jnp.where(lanes == lane, rv, 0))

            @pl.when(p == 0)
            def _():
                pltpu.async_copy(slab_v.at[0], neg_hbm.at[d], wsem0)

            @pl.when(p == 1)
            def _():
                pltpu.async_copy(slab_v.at[1], neg_hbm.at[d], wsem1)

            return carry2

        lax.fori_loop(0, cnt, dest_body, 0)

        @pl.when((j >= 1) & (p == 0))
        def _():
            pltpu.async_copy(slab_v.at[0], pos_hbm.at[j - 1], wsem0)

        @pl.when((j >= 1) & (p == 1))
        def _():
            pltpu.async_copy(slab_v.at[1], pos_hbm.at[j - 1], wsem1)

        writes = cnt + jnp.where(j >= 1, 1, 0)

        @pl.when(p == 0)
        def _():
            drain(wsem1, prev_writes)

        @pl.when(p == 1)
        def _():
            drain(wsem0, prev_writes)

        @pl.when(si + 1 < nslab)
        def _():
            jn = lane_of(wv, si + 2)

            @pl.when(p == 0)
            def _():
                pltpu.async_copy(x_hbm.at[jn], slab_v.at[1], lsem1)

            @pl.when(p == 1)
            def _():
                pltpu.async_copy(x_hbm.at[jn], slab_v.at[0], lsem0)

        return writes

    last_writes = lax.fori_loop(0, nslab, slab_body, 0)

    @pl.when((nslab % 2) == 1)
    def _():
        drain(wsem0, last_writes)

    @pl.when((nslab % 2) == 0)
    def _():
        drain(wsem1, last_writes)


def kernel(inputs):
    b, l, d = inputs.shape
    x_t = jnp.transpose(inputs, (1, 2, 0))
    pos_t, neg_t = _sc_route(x_t, jnp.asarray(_DTBL), jnp.asarray(_WTBL))
    pos = jnp.transpose(pos_t, (2, 0, 1))
    neg = jnp.transpose(
        neg_t.reshape(_LM1, _SAMPLE_NUM, d, b), (3, 0, 1, 2)
    )
    return pos, neg
